# baseline (speedup 1.0000x reference)
import jax
import jax.numpy as jnp
from jax.experimental import pallas as pl

_VB = 1024


def _onehot_block(idx_ref, out_ref):
    j = pl.program_id(0)
    idx = idx_ref[:, :]
    col = jax.lax.broadcasted_iota(jnp.int32, out_ref.shape, 1) + j * _VB
    out_ref[:, :] = (col == idx).astype(jnp.float32)


def kernel(indexes, weight):
    vocab = weight.shape[0]
    batch = indexes.shape[0]
    idx2 = indexes.astype(jnp.int32).reshape(batch, 1)
    return pl.pallas_call(
        _onehot_block,
        out_shape=jax.ShapeDtypeStruct((batch, vocab), jnp.float32),
        grid=(pl.cdiv(vocab, _VB),),
        in_specs=[pl.BlockSpec((batch, 1), lambda j: (0, 0))],
        out_specs=pl.BlockSpec((batch, _VB), lambda j: (0, j)),
    )(idx2)


# speedup vs baseline: 1.0024x; 1.0024x over previous
import jax
import jax.numpy as jnp
from jax.experimental import pallas as pl
from jax.experimental.pallas import tpu as pltpu

_VB = 2048


def _onehot_block(idx_ref, out_ref):
    j = pl.program_id(0)
    idx = idx_ref[:, :] - j * _VB
    col = jax.lax.broadcasted_iota(jnp.int32, out_ref.shape, 1)
    out_ref[:, :] = (col == idx).astype(jnp.float32)


def kernel(indexes, weight):
    vocab = weight.shape[0]
    batch = indexes.shape[0]
    idx2 = indexes.astype(jnp.int32).reshape(batch, 1)
    return pl.pallas_call(
        _onehot_block,
        out_shape=jax.ShapeDtypeStruct((batch, vocab), jnp.float32),
        grid=(pl.cdiv(vocab, _VB),),
        in_specs=[pl.BlockSpec((batch, 1), lambda j: (0, 0))],
        out_specs=pl.BlockSpec((batch, _VB), lambda j: (0, j)),
        compiler_params=pltpu.CompilerParams(
            dimension_semantics=("parallel",),
        ),
    )(idx2)
